# baseline (speedup 1.0000x reference)
import functools

import jax
import jax.numpy as jnp
from jax.experimental import pallas as pl
from jax.experimental.pallas import tpu as pltpu

_N = 2048
_D = 256
_P = 16
_EPS = 0.1
_K = 30

_BN = 512
_LW = 128
_R = 7


def _build_z_kernel(x_ref, w_ref, z_ref):
    x = x_ref[...]
    w = w_ref[...]
    y = x[:, None, :] * w[None, :, :]
    ss = jnp.sum(y * y, axis=-1, keepdims=True)
    norm = jnp.maximum(jnp.sqrt(ss), 1e-12)
    z = (y / norm) * 0.25
    z_ref[...] = z.reshape(x.shape[0], _P * _D).astype(jnp.bfloat16)


def _topk_filter(adj):
    ncols = adj.shape[1]
    nch = ncols // _LW

    work = []
    c = None
    for k in range(nch):
        s = adj[:, k * _LW:(k + 1) * _LW]
        s = jnp.where(s > _EPS, s, 0.0)
        work.append(s)
        c = s if c is None else jnp.maximum(c, s)

    planes = []
    for r in range(_R):
        planes.append(c)
        nxt = None
        for k in range(nch):
            s = jnp.where(work[k] == c, 0.0, work[k])
            work[k] = s
            if r < _R - 1:
                nxt = s if nxt is None else jnp.maximum(nxt, s)
        c = nxt

    tplanes = [p.T for p in planes]
    thresh = None
    for _ in range(_K):
        m = tplanes[0]
        for p in tplanes[1:]:
            m = jnp.maximum(m, p)
        m = jnp.max(m, axis=0, keepdims=True)
        tplanes = [jnp.where(p == m, 0.0, p) for p in tplanes]
        thresh = m

    return jnp.where((adj >= thresh.T) & (adj > _EPS), adj, 0.0)


def _adj_topk_kernel(b_ref, out_ref):
    i = pl.program_id(0)
    a = b_ref[pl.ds(i * _BN, _BN), :]
    adj = jax.lax.dot_general(
        a, b_ref[...], (((1,), (1,)), ((), ())),
        preferred_element_type=jnp.float32)
    out_ref[...] = _topk_filter(adj)


@jax.jit
def kernel(node_features, W):
    n, d = node_features.shape
    p = W.shape[0]
    pd = p * d
    nblk = n // _BN

    z = pl.pallas_call(
        _build_z_kernel,
        grid=(nblk,),
        in_specs=[
            pl.BlockSpec((_BN, d), lambda i: (i, 0)),
            pl.BlockSpec((p, d), lambda i: (0, 0)),
        ],
        out_specs=pl.BlockSpec((_BN, pd), lambda i: (i, 0)),
        out_shape=jax.ShapeDtypeStruct((n, pd), jnp.bfloat16),
    )(node_features, W)

    out = pl.pallas_call(
        _adj_topk_kernel,
        grid=(nblk,),
        in_specs=[
            pl.BlockSpec((n, pd), lambda i: (0, 0)),
        ],
        out_specs=pl.BlockSpec((_BN, n), lambda i: (i, 0)),
        out_shape=jax.ShapeDtypeStruct((n, n), jnp.float32),
    )(z)
    return out


# speedup vs baseline: 1.1385x; 1.1385x over previous
import functools

import jax
import jax.numpy as jnp
from jax.experimental import pallas as pl
from jax.experimental.pallas import tpu as pltpu

_N = 2048
_D = 256
_P = 16
_EPS = 0.1
_K = 30

_BN = 512
_ZB = 512
_LW = 128
_R = 7


def _topk_filter(adj):
    ncols = adj.shape[1]
    nch = ncols // _LW

    work = []
    c = None
    for k in range(nch):
        s = adj[:, k * _LW:(k + 1) * _LW]
        s = jnp.where(s > _EPS, s, 0.0)
        work.append(s)
        c = s if c is None else jnp.maximum(c, s)

    planes = []
    for r in range(_R):
        planes.append(c)
        nxt = None
        for k in range(nch):
            s = jnp.where(work[k] == c, 0.0, work[k])
            work[k] = s
            if r < _R - 1:
                nxt = s if nxt is None else jnp.maximum(nxt, s)
        c = nxt

    thresh = None
    for _ in range(_K):
        m = planes[0]
        for p in planes[1:]:
            m = jnp.maximum(m, p)
        m = jnp.max(m, axis=1, keepdims=True)
        planes = [jnp.where(p == m, 0.0, p) for p in planes]
        thresh = m

    return jnp.where((adj >= thresh) & (adj > _EPS), adj, 0.0)


def _fused_kernel(x_ref, w_ref, out_ref, z_ref):
    i = pl.program_id(0)

    @pl.when(i == 0)
    def _build_z():
        w = w_ref[...]
        for blk in range(_N // _ZB):
            x = x_ref[pl.ds(blk * _ZB, _ZB), :]
            y = x[:, None, :] * w[None, :, :]
            ss = jnp.sum(y * y, axis=-1, keepdims=True)
            norm = jnp.maximum(jnp.sqrt(ss), 1e-12)
            z = (y / norm) * 0.25
            z_ref[pl.ds(blk * _ZB, _ZB), :] = (
                z.reshape(_ZB, _P * _D).astype(jnp.bfloat16))

    a = z_ref[pl.ds(i * _BN, _BN), :]
    adj = jax.lax.dot_general(
        a, z_ref[...], (((1,), (1,)), ((), ())),
        preferred_element_type=jnp.float32)
    out_ref[...] = _topk_filter(adj)


@jax.jit
def kernel(node_features, W):
    n, d = node_features.shape
    p = W.shape[0]
    pd = p * d
    nblk = n // _BN

    out = pl.pallas_call(
        _fused_kernel,
        grid=(nblk,),
        in_specs=[
            pl.BlockSpec((n, d), lambda i: (0, 0)),
            pl.BlockSpec((p, d), lambda i: (0, 0)),
        ],
        out_specs=pl.BlockSpec((_BN, n), lambda i: (i, 0)),
        out_shape=jax.ShapeDtypeStruct((n, n), jnp.float32),
        scratch_shapes=[pltpu.VMEM((n, pd), jnp.bfloat16)],
    )(node_features, W)
    return out
